# Initial kernel scaffold; baseline (speedup 1.0000x reference)
import jax
import jax.numpy as jnp
from jax.experimental import pallas as pl


def kernel(feat, edge_index, edge_weight, W, b):
    raise NotImplementedError("write your pallas kernel here")


# speedup vs baseline: 3.2162x; 3.2162x over previous
import functools

import jax
import jax.numpy as jnp
from jax import lax
from jax.experimental import pallas as pl
from jax.experimental.pallas import tpu as pltpu
from jax.experimental.pallas import tpu_sc as plsc

NC = 2
NS = 16
LANES = 16

CHUNK = 256
SUB = 128
NSUB = CHUNK // SUB


def _sc_segment_sum(feat, srcp, dst2d, wp, n_nodes, d, chunks_per_tile):
    mesh = plsc.VectorSubcoreMesh(
        core_axis_name="c", subcore_axis_name="s", num_cores=NC,
        num_subcores=NS)
    rows_per_tile = (n_nodes // NS) & ~7
    tail_rows = n_nodes - NS * rows_per_tile

    @functools.partial(
        pl.kernel,
        out_type=jax.ShapeDtypeStruct((NC, n_nodes, d), jnp.float32),
        mesh=mesh,
        scratch_types=[
            pltpu.VMEM_SHARED((n_nodes, d), jnp.float32),
            pltpu.VMEM((CHUNK,), jnp.int32),
            pltpu.VMEM((NSUB, SUB), jnp.int32),
            pltpu.VMEM((CHUNK,), jnp.float32),
            pltpu.VMEM((CHUNK, d), jnp.float32),
        ],
    )
    def k(feat_hbm, src_hbm, dst_hbm, w_hbm, out_hbm,
          acc, src_v, dst_v, w_v, rows_v):
        c = lax.axis_index("c")
        s = lax.axis_index("s")
        wid = c * NS + s

        def zero_row(e, _):
            for d16 in range(d // LANES):
                rows_v[e, pl.ds(d16 * LANES, LANES)] = jnp.zeros(
                    (LANES,), jnp.float32)
            return _
        lax.fori_loop(0, CHUNK, zero_row, None)
        base_row = s * rows_per_tile
        for i in range(rows_per_tile // CHUNK):
            pltpu.sync_copy(rows_v,
                            acc.at[pl.ds(base_row + i * CHUNK, CHUNK)])
        rem = rows_per_tile % CHUNK
        if rem:
            pltpu.sync_copy(
                rows_v.at[pl.ds(0, rem)],
                acc.at[pl.ds(base_row + rows_per_tile - rem, rem)])

        @pl.when(s == NS - 1)
        def _():
            pltpu.sync_copy(
                rows_v.at[pl.ds(0, tail_rows)],
                acc.at[pl.ds(NS * rows_per_tile, tail_rows)])

        plsc.subcore_barrier()

        def chunk_body(kk, _):
            ebase = (wid * chunks_per_tile + kk) * CHUNK
            pltpu.sync_copy(src_hbm.at[pl.ds(ebase, CHUNK)], src_v)
            pltpu.sync_copy(w_hbm.at[pl.ds(ebase, CHUNK)], w_v)
            rbase = (wid * chunks_per_tile + kk) * NSUB
            pltpu.sync_copy(dst_hbm.at[pl.ds(rbase, NSUB)], dst_v)
            for j in range(NSUB):
                pltpu.sync_copy(
                    feat_hbm.at[src_v.at[pl.ds(j * SUB, SUB)]],
                    rows_v.at[pl.ds(j * SUB, SUB)])

            dnums = lax.GatherDimensionNumbers(
                offset_dims=(), collapsed_slice_dims=(0,),
                start_index_map=(0,))

            def mul_body(g, _):
                w16 = w_v[pl.ds(g * LANES, LANES)]
                for r in range(LANES):
                    wvec = lax.gather(
                        w16, jnp.full((LANES, 1), r, jnp.int32), dnums,
                        (1,), mode=lax.GatherScatterMode.PROMISE_IN_BOUNDS)
                    e = g * LANES + r
                    for d16 in range(d // LANES):
                        sl = pl.ds(d16 * LANES, LANES)
                        rows_v[e, sl] = rows_v[e, sl] * wvec
                return _
            lax.fori_loop(0, CHUNK // LANES, mul_body, None)

            for j in range(NSUB):
                pltpu.sync_copy(rows_v.at[pl.ds(j * SUB, SUB)],
                                acc.at[dst_v.at[j]], add=True)
            return _
        lax.fori_loop(0, chunks_per_tile, chunk_body, None)

        plsc.subcore_barrier()
        pltpu.sync_copy(acc.at[pl.ds(base_row, rows_per_tile)],
                        out_hbm.at[c, pl.ds(base_row, rows_per_tile)])

        @pl.when(s == NS - 1)
        def _():
            pltpu.sync_copy(
                acc.at[pl.ds(NS * rows_per_tile, tail_rows)],
                out_hbm.at[c, pl.ds(NS * rows_per_tile, tail_rows)])

    return k(feat, srcp, dst2d, wp)


def _tc_linear(partials, W, b2d, n_nodes, d):
    blk = 1000

    def body(p_ref, w_ref, b_ref, o_ref):
        x = p_ref[0] + p_ref[1]
        y = lax.dot_general(x, w_ref[...], (((1,), (1,)), ((), ())),
                            preferred_element_type=jnp.float32)
        o_ref[...] = y + b_ref[...]

    return pl.pallas_call(
        body,
        grid=(n_nodes // blk,),
        in_specs=[
            pl.BlockSpec((NC, blk, d), lambda i: (0, i, 0)),
            pl.BlockSpec((d, d), lambda i: (0, 0)),
            pl.BlockSpec((1, d), lambda i: (0, 0)),
        ],
        out_specs=pl.BlockSpec((blk, d), lambda i: (i, 0)),
        out_shape=jax.ShapeDtypeStruct((n_nodes, d), jnp.float32),
    )(partials, W, b2d)


def kernel(feat, edge_index, edge_weight, W, b):
    n_nodes, d = feat.shape
    n_edges = edge_index.shape[1]

    edges_per_tile_pad = ((n_edges + NC * NS * CHUNK - 1)
                          // (NC * NS * CHUNK)) * CHUNK
    e_pad = NC * NS * edges_per_tile_pad
    pad = e_pad - n_edges
    src = jnp.concatenate(
        [edge_index[0], jnp.zeros((pad,), jnp.int32)])
    dst = jnp.concatenate(
        [edge_index[1], jnp.zeros((pad,), jnp.int32)])
    w = jnp.concatenate(
        [edge_weight, jnp.zeros((pad,), jnp.float32)])
    dst2d = dst.reshape(e_pad // SUB, SUB)

    partials = _sc_segment_sum(feat, src, dst2d, w, n_nodes, d,
                               edges_per_tile_pad // CHUNK)
    return _tc_linear(partials, W, b.reshape(1, d), n_nodes, d)


# speedup vs baseline: 5.7855x; 1.7989x over previous
import functools

import jax
import jax.numpy as jnp
from jax import lax
from jax.experimental import pallas as pl
from jax.experimental.pallas import tpu as pltpu
from jax.experimental.pallas import tpu_sc as plsc

NC = 2
NS = 16
LANES = 16

CHUNK = 256
SUB = 128
NSUB = CHUNK // SUB


def _sc_segment_sum(feat, srcp, dst2d, wp, n_nodes, d, chunks_per_tile):
    mesh = plsc.VectorSubcoreMesh(
        core_axis_name="c", subcore_axis_name="s", num_cores=NC,
        num_subcores=NS)
    rows_per_tile = (n_nodes // NS) & ~7
    tail_rows = n_nodes - NS * rows_per_tile

    @functools.partial(
        pl.kernel,
        out_type=jax.ShapeDtypeStruct((NC, n_nodes, d), jnp.float32),
        mesh=mesh,
        scratch_types=[
            pltpu.VMEM_SHARED((n_nodes, d), jnp.float32),
            pltpu.VMEM((CHUNK,), jnp.int32),
            pltpu.VMEM((NSUB, SUB), jnp.int32),
            pltpu.VMEM((CHUNK,), jnp.float32),
            pltpu.VMEM((CHUNK, d), jnp.float32),
        ],
    )
    def k(feat_hbm, src_hbm, dst_hbm, w_hbm, out_hbm,
          acc, src_v, dst_v, w_v, rows_v):
        c = lax.axis_index("c")
        s = lax.axis_index("s")
        wid = c * NS + s

        def zero_row(e, _):
            for d16 in range(d // LANES):
                rows_v[e, pl.ds(d16 * LANES, LANES)] = jnp.zeros(
                    (LANES,), jnp.float32)
            return _
        lax.fori_loop(0, CHUNK, zero_row, None)
        base_row = s * rows_per_tile
        for i in range(rows_per_tile // CHUNK):
            pltpu.sync_copy(rows_v,
                            acc.at[pl.ds(base_row + i * CHUNK, CHUNK)])
        rem = rows_per_tile % CHUNK
        if rem:
            pltpu.sync_copy(
                rows_v.at[pl.ds(0, rem)],
                acc.at[pl.ds(base_row + rows_per_tile - rem, rem)])

        @pl.when(s == NS - 1)
        def _():
            pltpu.sync_copy(
                rows_v.at[pl.ds(0, tail_rows)],
                acc.at[pl.ds(NS * rows_per_tile, tail_rows)])

        plsc.subcore_barrier()

        def chunk_body(kk, _):
            ebase = (wid * chunks_per_tile + kk) * CHUNK
            pltpu.sync_copy(src_hbm.at[pl.ds(ebase, CHUNK)], src_v)
            pltpu.sync_copy(w_hbm.at[pl.ds(ebase, CHUNK)], w_v)
            rbase = (wid * chunks_per_tile + kk) * NSUB
            pltpu.sync_copy(dst_hbm.at[pl.ds(rbase, NSUB)], dst_v)
            for j in range(NSUB):
                pltpu.sync_copy(
                    feat_hbm.at[src_v.at[pl.ds(j * SUB, SUB)]],
                    rows_v.at[pl.ds(j * SUB, SUB)])

            dnums = lax.GatherDimensionNumbers(
                offset_dims=(), collapsed_slice_dims=(0,),
                start_index_map=(0,))

            def mul_body(g, _):
                w16 = w_v[pl.ds(g * LANES, LANES)]
                for r in range(LANES):
                    wvec = lax.gather(
                        w16, jnp.full((LANES, 1), r, jnp.int32), dnums,
                        (1,), mode=lax.GatherScatterMode.PROMISE_IN_BOUNDS)
                    e = g * LANES + r
                    for d16 in range(d // LANES):
                        sl = pl.ds(d16 * LANES, LANES)
                        rows_v[e, sl] = rows_v[e, sl] * wvec
                return _
            lax.fori_loop(0, CHUNK // LANES, mul_body, None)

            for j in range(NSUB):
                pltpu.sync_copy(rows_v.at[pl.ds(j * SUB, SUB)],
                                acc.at[dst_v.at[j]], add=True)
            return _
        lax.fori_loop(0, chunks_per_tile, chunk_body, None)

        plsc.subcore_barrier()
        pltpu.sync_copy(acc.at[pl.ds(base_row, rows_per_tile)],
                        out_hbm.at[c, pl.ds(base_row, rows_per_tile)])

        @pl.when(s == NS - 1)
        def _():
            pltpu.sync_copy(
                acc.at[pl.ds(NS * rows_per_tile, tail_rows)],
                out_hbm.at[c, pl.ds(NS * rows_per_tile, tail_rows)])

    return k(feat, srcp, dst2d, wp)


def _tc_linear(partials, W, b2d, n_nodes, d):
    blk = 1000

    def body(p_ref, w_ref, b_ref, o_ref):
        x = p_ref[0] + p_ref[1]
        y = lax.dot_general(x, w_ref[...], (((1,), (1,)), ((), ())),
                            preferred_element_type=jnp.float32)
        o_ref[...] = y + b_ref[...]

    return pl.pallas_call(
        body,
        grid=(n_nodes // blk,),
        in_specs=[
            pl.BlockSpec((NC, blk, d), lambda i: (0, i, 0)),
            pl.BlockSpec((d, d), lambda i: (0, 0)),
            pl.BlockSpec((1, d), lambda i: (0, 0)),
        ],
        out_specs=pl.BlockSpec((blk, d), lambda i: (i, 0)),
        out_shape=jax.ShapeDtypeStruct((n_nodes, d), jnp.float32),
    )(partials, W, b2d)


def kernel(feat, edge_index, edge_weight, W, b):
    n_nodes, d = feat.shape
    n_edges = edge_index.shape[1]

    edges_per_tile_pad = ((n_edges + NC * NS * CHUNK - 1)
                          // (NC * NS * CHUNK)) * CHUNK
    e_pad = NC * NS * edges_per_tile_pad
    pad = e_pad - n_edges
    spread = jnp.arange(pad, dtype=jnp.int32) % n_nodes
    src = jnp.concatenate([edge_index[0], spread])
    dst = jnp.concatenate([edge_index[1], spread])
    w = jnp.concatenate(
        [edge_weight, jnp.zeros((pad,), jnp.float32)])
    dst2d = dst.reshape(e_pad // SUB, SUB)

    partials = _sc_segment_sum(feat, src, dst2d, w, n_nodes, d,
                               edges_per_tile_pad // CHUNK)
    return _tc_linear(partials, W, b.reshape(1, d), n_nodes, d)
